# baseline (speedup 1.0000x reference)
import jax
import jax.numpy as jnp
from jax import lax
from jax.experimental import pallas as pl
from jax.experimental.pallas import tpu as pltpu
from jax.experimental.pallas import tpu_sc as plsc

NUM_ROWS = 1000000
NBLK = 31
NPROW = NBLK * 16384
BATCH = 16384
DIM = 64
TCC = 32768
WINDOW = 128
LANES = 16

_MESH = plsc.VectorSubcoreMesh(core_axis_name="core",
                               subcore_axis_name="subcore")
_SC_PARAMS = pltpu.CompilerParams(use_tc_tiling_on_sc=False,
                                  needs_layout_passes=False)


def _sc_scratch():
    return [
        pltpu.VMEM((WINDOW, DIM), jnp.int32),
        pltpu.VMEM((WINDOW,), jnp.int32),
        pltpu.VMEM((WINDOW,), jnp.int32),
        pltpu.SemaphoreType.DMA,
    ]


def _tc_transpose(tT):
    def body(in_ref, out_ref):
        row = lax.broadcasted_iota(jnp.int32, (DIM, DIM), 0)
        col = lax.broadcasted_iota(jnp.int32, (DIM, DIM), 1)
        ident = jnp.where(row == col, 1.0, 0.0).astype(jnp.float32)
        t = lax.dot_general(in_ref[...], ident, (((0,), (0,)), ((), ())),
                            precision=lax.Precision.DEFAULT)
        tb = t.astype(jnp.bfloat16)
        out_ref[...] = jnp.concatenate([tb[:TCC // 2], tb[TCC // 2:]], axis=1)

    return pl.pallas_call(
        body,
        grid=(pl.cdiv(NUM_ROWS, TCC),),
        in_specs=[pl.BlockSpec((DIM, TCC), lambda i: (0, i))],
        out_specs=pl.BlockSpec((TCC // 2, 2 * DIM), lambda i: (i, 0)),
        out_shape=jax.ShapeDtypeStruct((NPROW, 2 * DIM), jnp.bfloat16),
        compiler_params=pltpu.CompilerParams(
            dimension_semantics=("parallel",)),
    )(tT)


def _idx_split(idx_vec, pid, hoff, c):
    v = idx_vec.at[0][pl.ds(c, LANES)]
    pid.at[pl.ds(c, LANES)][...] = (
        lax.shift_left(lax.shift_right_logical(v, 15), 14)
        + lax.bitwise_and(v, 16383))
    hoff.at[pl.ds(c, LANES)][...] = (
        lax.bitwise_and(lax.shift_right_logical(v, 14), 1) * (DIM // 2))


def _sc_gather_u(idx2, tab2):
    @pl.kernel(
        out_type=jax.ShapeDtypeStruct((DIM, BATCH), jnp.float32),
        mesh=_MESH,
        compiler_params=_SC_PARAMS,
        scratch_types=_sc_scratch(),
    )
    def sc_kernel(u_hbm, t_hbm, o_hbm, gbuf, pid, hoff, sem):
        def body(u_idx, o_vmem):
            @pl.loop(0, WINDOW, step=LANES)
            def _(c):
                _idx_split(u_idx, pid, hoff, c)

            cp = pltpu.make_async_copy(t_hbm.at[pid], gbuf, sem)
            cp.start()
            cp.wait()

            @pl.loop(0, DIM)
            def _(d):
                @pl.loop(0, WINDOW, step=LANES)
                def _(c):
                    rows = lax.iota(jnp.int32, LANES) + c
                    cols = (hoff.at[pl.ds(c, LANES)][...]
                            + lax.shift_right_logical(d, 1))
                    raw = plsc.load_gather(gbuf, [rows, cols])
                    shifted = lax.shift_left(
                        raw, (1 - lax.bitwise_and(d, 1)) * 16)
                    bits = lax.bitwise_and(shifted, jnp.int32(-65536))
                    o_vmem.at[d][pl.ds(c, LANES)] = plsc.bitcast(
                        bits, jnp.float32)

        pltpu.emit_pipeline(
            body,
            grid=(BATCH // WINDOW,),
            in_specs=[pl.BlockSpec((1, WINDOW), lambda i: (0, i))],
            out_specs=[pl.BlockSpec((DIM, WINDOW), lambda i: (0, i))],
            core_axis_name=("core", "subcore"),
            dimension_semantics=(pltpu.PARALLEL,),
        )(u_hbm, o_hbm)

    return sc_kernel(idx2, tab2)


def _sc_gather_i_mul(idx2, tab2, gu):
    @pl.kernel(
        out_type=jax.ShapeDtypeStruct((DIM, BATCH), jnp.float32),
        mesh=_MESH,
        compiler_params=_SC_PARAMS,
        scratch_types=_sc_scratch(),
    )
    def sc_kernel(i_hbm, t_hbm, g_hbm, o_hbm, gbuf, pid, hoff, sem):
        def body(i_idx, g_blk, o_vmem):
            @pl.loop(0, WINDOW, step=LANES)
            def _(c):
                _idx_split(i_idx, pid, hoff, c)

            cp = pltpu.make_async_copy(t_hbm.at[pid], gbuf, sem)
            cp.start()
            cp.wait()

            @pl.loop(0, DIM)
            def _(d):
                @pl.loop(0, WINDOW, step=LANES)
                def _(c):
                    rows = lax.iota(jnp.int32, LANES) + c
                    cols = (hoff.at[pl.ds(c, LANES)][...]
                            + lax.shift_right_logical(d, 1))
                    raw = plsc.load_gather(gbuf, [rows, cols])
                    shifted = lax.shift_left(
                        raw, (1 - lax.bitwise_and(d, 1)) * 16)
                    bits = lax.bitwise_and(shifted, jnp.int32(-65536))
                    ivals = plsc.bitcast(bits, jnp.float32)
                    o_vmem.at[d][pl.ds(c, LANES)] = (
                        ivals * g_blk.at[d][pl.ds(c, LANES)])

        pltpu.emit_pipeline(
            body,
            grid=(BATCH // WINDOW,),
            in_specs=[
                pl.BlockSpec((1, WINDOW), lambda i: (0, i)),
                pl.BlockSpec((DIM, WINDOW), lambda i: (0, i)),
            ],
            out_specs=[pl.BlockSpec((DIM, WINDOW), lambda i: (0, i))],
            core_axis_name=("core", "subcore"),
            dimension_semantics=(pltpu.PARALLEL,),
        )(i_hbm, g_hbm, o_hbm)

    return sc_kernel(idx2, tab2, gu)


def kernel(user, item, user_table, item_table):
    ut2 = lax.bitcast_convert_type(
        _tc_transpose(user_table.T).reshape(NPROW, DIM, 2), jnp.int32)
    it2 = lax.bitcast_convert_type(
        _tc_transpose(item_table.T).reshape(NPROW, DIM, 2), jnp.int32)
    u2 = user.reshape(1, BATCH)
    i2 = item.reshape(1, BATCH)

    gu = _sc_gather_u(u2, ut2)
    out = _sc_gather_i_mul(i2, it2, gu)
    return out.T


# speedup vs baseline: 6.8843x; 6.8843x over previous
import jax
import jax.numpy as jnp
from jax import lax
from jax.experimental import pallas as pl
from jax.experimental.pallas import tpu as pltpu
from jax.experimental.pallas import tpu_sc as plsc

NUM_ROWS = 1000000
NBLK = 31
NPROW = NBLK * 16384
BATCH = 16384
DIM = 64
TCC = 32768
WINDOW = 128
LANES = 16

_MESH = plsc.VectorSubcoreMesh(core_axis_name="core",
                               subcore_axis_name="subcore")
_SC_PARAMS = pltpu.CompilerParams(use_tc_tiling_on_sc=False,
                                  needs_layout_passes=False)


def _sc_scratch():
    return [
        pltpu.VMEM((WINDOW, 2 * DIM), jnp.float32),
        pltpu.VMEM((WINDOW,), jnp.int32),
        pltpu.VMEM((WINDOW,), jnp.int32),
        pltpu.SemaphoreType.DMA,
    ]


def _tc_transpose(tT):
    def body(in_ref, out_ref):
        row = lax.broadcasted_iota(jnp.int32, (DIM, DIM), 0)
        col = lax.broadcasted_iota(jnp.int32, (DIM, DIM), 1)
        ident = jnp.where(row == col, 1.0, 0.0).astype(jnp.float32)
        t = lax.dot_general(in_ref[...], ident, (((0,), (0,)), ((), ())),
                            precision=lax.Precision.DEFAULT)
        out_ref[...] = jnp.concatenate([t[:TCC // 2], t[TCC // 2:]], axis=1)

    return pl.pallas_call(
        body,
        grid=(pl.cdiv(NUM_ROWS, TCC),),
        in_specs=[pl.BlockSpec((DIM, TCC), lambda i: (0, i))],
        out_specs=pl.BlockSpec((TCC // 2, 2 * DIM), lambda i: (i, 0)),
        out_shape=jax.ShapeDtypeStruct((NPROW, 2 * DIM), jnp.float32),
        compiler_params=pltpu.CompilerParams(
            dimension_semantics=("parallel",)),
    )(tT)


def _idx_split(idx_vec, pid, hoff, c):
    v = idx_vec.at[0][pl.ds(c, LANES)]
    pid.at[pl.ds(c, LANES)][...] = (
        lax.shift_left(lax.shift_right_logical(v, 15), 14)
        + lax.bitwise_and(v, 16383))
    hoff.at[pl.ds(c, LANES)][...] = (
        lax.bitwise_and(lax.shift_right_logical(v, 14), 1) * DIM)


def _sc_gather_u(idx2, tab2):
    @pl.kernel(
        out_type=jax.ShapeDtypeStruct((DIM, BATCH), jnp.float32),
        mesh=_MESH,
        compiler_params=_SC_PARAMS,
        scratch_types=_sc_scratch(),
    )
    def sc_kernel(u_hbm, t_hbm, o_hbm, gbuf, pid, hoff, sem):
        def body(u_idx, o_vmem):
            @pl.loop(0, WINDOW, step=LANES)
            def _(c):
                _idx_split(u_idx, pid, hoff, c)

            cp = pltpu.make_async_copy(t_hbm.at[pid], gbuf, sem)
            cp.start()
            cp.wait()

            @pl.loop(0, DIM)
            def _(d):
                @pl.loop(0, WINDOW, step=LANES)
                def _(c):
                    rows = lax.iota(jnp.int32, LANES) + c
                    cols = hoff.at[pl.ds(c, LANES)][...] + d
                    o_vmem.at[d][pl.ds(c, LANES)] = plsc.load_gather(
                        gbuf, [rows, cols])

        pltpu.emit_pipeline(
            body,
            grid=(BATCH // WINDOW,),
            in_specs=[pl.BlockSpec((1, WINDOW), lambda i: (0, i))],
            out_specs=[pl.BlockSpec((DIM, WINDOW), lambda i: (0, i))],
            core_axis_name=("core", "subcore"),
            dimension_semantics=(pltpu.PARALLEL,),
        )(u_hbm, o_hbm)

    return sc_kernel(idx2, tab2)


def _sc_gather_i_mul(idx2, tab2, gu):
    @pl.kernel(
        out_type=jax.ShapeDtypeStruct((DIM, BATCH), jnp.float32),
        mesh=_MESH,
        compiler_params=_SC_PARAMS,
        scratch_types=_sc_scratch(),
    )
    def sc_kernel(i_hbm, t_hbm, g_hbm, o_hbm, gbuf, pid, hoff, sem):
        def body(i_idx, g_blk, o_vmem):
            @pl.loop(0, WINDOW, step=LANES)
            def _(c):
                _idx_split(i_idx, pid, hoff, c)

            cp = pltpu.make_async_copy(t_hbm.at[pid], gbuf, sem)
            cp.start()
            cp.wait()

            @pl.loop(0, DIM)
            def _(d):
                @pl.loop(0, WINDOW, step=LANES)
                def _(c):
                    rows = lax.iota(jnp.int32, LANES) + c
                    cols = hoff.at[pl.ds(c, LANES)][...] + d
                    ivals = plsc.load_gather(gbuf, [rows, cols])
                    o_vmem.at[d][pl.ds(c, LANES)] = (
                        ivals * g_blk.at[d][pl.ds(c, LANES)])

        pltpu.emit_pipeline(
            body,
            grid=(BATCH // WINDOW,),
            in_specs=[
                pl.BlockSpec((1, WINDOW), lambda i: (0, i)),
                pl.BlockSpec((DIM, WINDOW), lambda i: (0, i)),
            ],
            out_specs=[pl.BlockSpec((DIM, WINDOW), lambda i: (0, i))],
            core_axis_name=("core", "subcore"),
            dimension_semantics=(pltpu.PARALLEL,),
        )(i_hbm, g_hbm, o_hbm)

    return sc_kernel(idx2, tab2, gu)


def kernel(user, item, user_table, item_table):
    ut2 = _tc_transpose(user_table.T)
    it2 = _tc_transpose(item_table.T)
    u2 = user.reshape(1, BATCH)
    i2 = item.reshape(1, BATCH)

    gu = _sc_gather_u(u2, ut2)
    out = _sc_gather_i_mul(i2, it2, gu)
    return out.T


# speedup vs baseline: 6.9028x; 1.0027x over previous
import jax
import jax.numpy as jnp
from jax import lax
from jax.experimental import pallas as pl
from jax.experimental.pallas import tpu as pltpu
from jax.experimental.pallas import tpu_sc as plsc

NUM_ROWS = 1000000
NBLK = 31
NPROW = NBLK * 16384
BATCH = 16384
DIM = 64
TCC = 32768
WINDOW = 256
LANES = 16

_MESH = plsc.VectorSubcoreMesh(core_axis_name="core",
                               subcore_axis_name="subcore")
_SC_PARAMS = pltpu.CompilerParams(use_tc_tiling_on_sc=False,
                                  needs_layout_passes=False)


def _sc_scratch():
    return [
        pltpu.VMEM((WINDOW, 2 * DIM), jnp.float32),
        pltpu.VMEM((WINDOW,), jnp.int32),
        pltpu.VMEM((WINDOW,), jnp.int32),
        pltpu.SemaphoreType.DMA,
    ]


def _tc_transpose(tT):
    def body(in_ref, out_ref):
        row = lax.broadcasted_iota(jnp.int32, (DIM, DIM), 0)
        col = lax.broadcasted_iota(jnp.int32, (DIM, DIM), 1)
        ident = jnp.where(row == col, 1.0, 0.0).astype(jnp.float32)
        t = lax.dot_general(in_ref[...], ident, (((0,), (0,)), ((), ())),
                            precision=lax.Precision.DEFAULT)
        out_ref[...] = jnp.concatenate([t[:TCC // 2], t[TCC // 2:]], axis=1)

    return pl.pallas_call(
        body,
        grid=(pl.cdiv(NUM_ROWS, TCC),),
        in_specs=[pl.BlockSpec((DIM, TCC), lambda i: (0, i))],
        out_specs=pl.BlockSpec((TCC // 2, 2 * DIM), lambda i: (i, 0)),
        out_shape=jax.ShapeDtypeStruct((NPROW, 2 * DIM), jnp.float32),
        compiler_params=pltpu.CompilerParams(
            dimension_semantics=("parallel",)),
    )(tT)


def _idx_split(idx_vec, pid, hoff, c):
    v = idx_vec.at[0][pl.ds(c, LANES)]
    pid.at[pl.ds(c, LANES)][...] = (
        lax.shift_left(lax.shift_right_logical(v, 15), 14)
        + lax.bitwise_and(v, 16383))
    hoff.at[pl.ds(c, LANES)][...] = (
        lax.bitwise_and(lax.shift_right_logical(v, 14), 1) * DIM)


def _sc_gather_u(idx2, tab2):
    @pl.kernel(
        out_type=jax.ShapeDtypeStruct((DIM, BATCH), jnp.float32),
        mesh=_MESH,
        compiler_params=_SC_PARAMS,
        scratch_types=_sc_scratch(),
    )
    def sc_kernel(u_hbm, t_hbm, o_hbm, gbuf, pid, hoff, sem):
        def body(u_idx, o_vmem):
            @pl.loop(0, WINDOW, step=LANES)
            def _(c):
                _idx_split(u_idx, pid, hoff, c)

            cp = pltpu.make_async_copy(t_hbm.at[pid], gbuf, sem)
            cp.start()
            cp.wait()

            @pl.loop(0, DIM)
            def _(d):
                @pl.loop(0, WINDOW, step=LANES)
                def _(c):
                    rows = lax.iota(jnp.int32, LANES) + c
                    cols = hoff.at[pl.ds(c, LANES)][...] + d
                    o_vmem.at[d][pl.ds(c, LANES)] = plsc.load_gather(
                        gbuf, [rows, cols])

        pltpu.emit_pipeline(
            body,
            grid=(BATCH // WINDOW,),
            in_specs=[pl.BlockSpec((1, WINDOW), lambda i: (0, i))],
            out_specs=[pl.BlockSpec((DIM, WINDOW), lambda i: (0, i))],
            core_axis_name=("core", "subcore"),
            dimension_semantics=(pltpu.PARALLEL,),
        )(u_hbm, o_hbm)

    return sc_kernel(idx2, tab2)


def _sc_gather_i_mul(idx2, tab2, gu):
    @pl.kernel(
        out_type=jax.ShapeDtypeStruct((DIM, BATCH), jnp.float32),
        mesh=_MESH,
        compiler_params=_SC_PARAMS,
        scratch_types=_sc_scratch(),
    )
    def sc_kernel(i_hbm, t_hbm, g_hbm, o_hbm, gbuf, pid, hoff, sem):
        def body(i_idx, g_blk, o_vmem):
            @pl.loop(0, WINDOW, step=LANES)
            def _(c):
                _idx_split(i_idx, pid, hoff, c)

            cp = pltpu.make_async_copy(t_hbm.at[pid], gbuf, sem)
            cp.start()
            cp.wait()

            @pl.loop(0, DIM)
            def _(d):
                @pl.loop(0, WINDOW, step=LANES)
                def _(c):
                    rows = lax.iota(jnp.int32, LANES) + c
                    cols = hoff.at[pl.ds(c, LANES)][...] + d
                    ivals = plsc.load_gather(gbuf, [rows, cols])
                    o_vmem.at[d][pl.ds(c, LANES)] = (
                        ivals * g_blk.at[d][pl.ds(c, LANES)])

        pltpu.emit_pipeline(
            body,
            grid=(BATCH // WINDOW,),
            in_specs=[
                pl.BlockSpec((1, WINDOW), lambda i: (0, i)),
                pl.BlockSpec((DIM, WINDOW), lambda i: (0, i)),
            ],
            out_specs=[pl.BlockSpec((DIM, WINDOW), lambda i: (0, i))],
            core_axis_name=("core", "subcore"),
            dimension_semantics=(pltpu.PARALLEL,),
        )(i_hbm, g_hbm, o_hbm)

    return sc_kernel(idx2, tab2, gu)


def kernel(user, item, user_table, item_table):
    ut2 = _tc_transpose(user_table.T)
    it2 = _tc_transpose(item_table.T)
    u2 = user.reshape(1, BATCH)
    i2 = item.reshape(1, BATCH)

    gu = _sc_gather_u(u2, ut2)
    out = _sc_gather_i_mul(i2, it2, gu)
    return out.T
